# baseline (speedup 1.0000x reference)
import functools

import jax
import jax.numpy as jnp
from jax import lax
from jax.experimental import pallas as pl
from jax.experimental.pallas import tpu as pltpu
from jax.experimental.pallas import tpu_sc as plsc

N_TYPES = 3
EMB_D = 64
QUAD = 4
QD = EMB_D * QUAD
QCHUNK = 128


@functools.lru_cache(maxsize=None)
def _make_lookup(bq: int):
    info = plsc.get_sparse_core_info()
    nw = info.num_cores * info.num_subcores
    per_w = bq // nw
    n_chunks = per_w // QCHUNK
    assert bq % (nw * QCHUNK) == 0 and n_chunks % 2 == 0

    mesh = plsc.VectorSubcoreMesh(core_axis_name="c", subcore_axis_name="s")

    @functools.partial(
        pl.kernel,
        mesh=mesh,
        out_type=jax.ShapeDtypeStruct((bq, QD), jnp.float32),
        scratch_types=[
            pltpu.VMEM((n_chunks, QCHUNK), jnp.int32),
            pltpu.VMEM((QCHUNK, QD), jnp.float32),
            pltpu.VMEM((QCHUNK, QD), jnp.float32),
            pltpu.SemaphoreType.DMA,
            pltpu.SemaphoreType.DMA,
            pltpu.SemaphoreType.DMA,
            pltpu.SemaphoreType.DMA,
        ],
    )
    def lookup(qtbl_hbm, qid_hbm, out_hbm, qid_v, rows0, rows1,
               gsem0, gsem1, osem0, osem1):
        wid = lax.axis_index("s") * info.num_cores + lax.axis_index("c")
        base0 = wid * per_w

        pltpu.sync_copy(qid_hbm.at[pl.ds(wid * n_chunks, n_chunks)], qid_v)

        def gather(c, rows, gsem):
            pltpu.async_copy(qtbl_hbm.at[qid_v.at[c]], rows, gsem)

        def wait_gather(c, rows, gsem):
            pltpu.make_async_copy(qtbl_hbm.at[qid_v.at[c]], rows, gsem).wait()

        def put(c, rows, osem):
            pltpu.async_copy(
                rows, out_hbm.at[pl.ds(base0 + c * QCHUNK, QCHUNK)], osem)

        def wait_put(c, rows, osem):
            pltpu.make_async_copy(
                rows, out_hbm.at[pl.ds(base0 + c * QCHUNK, QCHUNK)], osem).wait()

        gather(0, rows0, gsem0)
        gather(1, rows1, gsem1)

        def body(j, carry):
            c0 = 2 * j
            wait_gather(c0, rows0, gsem0)
            put(c0, rows0, osem0)
            wait_gather(c0 + 1, rows1, gsem1)
            put(c0 + 1, rows1, osem1)
            wait_put(c0, rows0, osem0)

            @pl.when(c0 + 2 < n_chunks)
            def _():
                gather(c0 + 2, rows0, gsem0)

            wait_put(c0 + 1, rows1, osem1)

            @pl.when(c0 + 3 < n_chunks)
            def _():
                gather(c0 + 3, rows1, gsem1)

            return carry

        lax.fori_loop(0, n_chunks // 2, body, 0)

    return lookup


def kernel(type_indices, embedding_weight):
    b, s = type_indices.shape
    quads = type_indices.reshape(b * s // QUAD, QUAD).astype(jnp.int32)
    weights = jnp.array([N_TYPES**(QUAD - 1 - k) for k in range(QUAD)], jnp.int32)
    qid = (quads @ weights).reshape(-1, QCHUNK)
    q = jnp.arange(N_TYPES**QUAD)
    digits = jnp.stack(
        [(q // (N_TYPES**(QUAD - 1 - k))) % N_TYPES for k in range(QUAD)], axis=-1
    )
    qtbl = embedding_weight[digits].reshape(N_TYPES**QUAD, QD)
    out = _make_lookup(b * s // QUAD)(qtbl, qid)
    return out.reshape(b, s, EMB_D)
